# Initial kernel scaffold; baseline (speedup 1.0000x reference)
import jax
import jax.numpy as jnp
from jax.experimental import pallas as pl


def kernel(x, w, b, edge_src, edge_dst):
    raise NotImplementedError("write your pallas kernel here")


# speedup vs baseline: 4.5376x; 4.5376x over previous
import functools

import jax
import jax.numpy as jnp
from jax import lax
from jax.experimental import pallas as pl
from jax.experimental.pallas import tpu as pltpu
from jax.experimental.pallas import tpu_sc as plsc

B = 1024
NPL = 1024
L = 8
DEG = 16
EPL = NPL * DEG
NW = 32
BPW = B // NW
WBLK = NPL * BPW


def _sc_forward(x_flat, srcs, ws, bs):
    mesh = plsc.VectorSubcoreMesh(core_axis_name="c", subcore_axis_name="s")

    @functools.partial(
        pl.kernel,
        mesh=mesh,
        out_type=jax.ShapeDtypeStruct((NW * WBLK,), jnp.float32),
        scratch_types=[
            pltpu.VMEM((WBLK,), jnp.float32),
            pltpu.VMEM((WBLK,), jnp.float32),
            pltpu.VMEM((EPL,), jnp.int32),
            pltpu.VMEM((EPL,), jnp.float32),
            pltpu.VMEM((NPL + 16,), jnp.float32),
        ],
    )
    def body(x_hbm, srcs_hbm, ws_hbm, bs_hbm, out_hbm, acts_a, acts_b, src_v,
             w_v, b_v):
        wid = lax.axis_index("s") * 2 + lax.axis_index("c")
        pltpu.sync_copy(x_hbm.at[pl.ds(wid * WBLK, WBLK)], acts_a)

        bufs = [acts_a, acts_b]
        for l in range(L - 1):
            cur = bufs[l % 2]
            nxt = bufs[(l + 1) % 2]
            pltpu.sync_copy(srcs_hbm.at[pl.ds(l * EPL, EPL)], src_v)
            pltpu.sync_copy(ws_hbm.at[pl.ds(l * EPL, EPL)], w_v)
            pltpu.sync_copy(bs_hbm.at[pl.ds(l * NPL, NPL)],
                            b_v.at[pl.ds(0, NPL)])

            def node_body(n, _, cur=cur, nxt=nxt):
                e0 = n * DEG
                bn = b_v[pl.ds(n, 16)][0]
                src16 = src_v[pl.ds(e0, DEG)]
                w16 = w_v[pl.ds(e0, DEG)]
                acc0 = jnp.full((16,), bn, jnp.float32)
                acc1 = jnp.full((16,), bn, jnp.float32)
                for k in range(DEG):
                    r0 = src16[k] * BPW
                    wk = w16[k]
                    acc0 = acc0 + wk * cur[pl.ds(r0, 16)]
                    acc1 = acc1 + wk * cur[pl.ds(r0 + 16, 16)]
                acc0 = jnp.maximum(acc0, 0.1 * acc0)
                acc1 = jnp.maximum(acc1, 0.1 * acc1)
                o0 = n * BPW
                nxt[pl.ds(o0, 16)] = acc0
                nxt[pl.ds(o0 + 16, 16)] = acc1
                return 0

            lax.fori_loop(0, NPL, node_body, 0)

        pltpu.sync_copy(bufs[(L - 1) % 2],
                        out_hbm.at[pl.ds(wid * WBLK, WBLK)])

    return body(x_flat, srcs, ws, bs)


def kernel(x, w, b, edge_src, edge_dst):
    del edge_dst
    srcs = (edge_src.reshape(L - 1, EPL) - (
        jnp.arange(L - 1, dtype=jnp.int32) * NPL)[:, None]).reshape(-1)
    bs = b[NPL:]
    x_flat = x.T.reshape(NPL, NW, BPW).transpose(1, 0, 2).reshape(-1)
    out_flat = _sc_forward(x_flat, srcs, w, bs)
    return out_flat.reshape(NW, NPL, BPW).transpose(1, 0, 2).reshape(NPL, B).T
